# baseline (speedup 1.0000x reference)
import jax
import jax.numpy as jnp
from jax.experimental import pallas as pl

_TN = 512


def _chamfer_body(x1_ref, x2t_ref, d1_ref, d2_ref):
    i = pl.program_id(1)
    x1 = x1_ref[0]
    x2t = x2t_ref[0]
    sq1 = jnp.sum(x1 * x1, axis=1, keepdims=True)
    sq2 = jnp.sum(x2t * x2t, axis=0, keepdims=True)
    inner = jax.lax.dot_general(
        x1, x2t, (((1,), (0,)), ((), ())),
        preferred_element_type=jnp.float32)
    acc = jnp.maximum(sq1 + sq2 - 2.0 * inner, 0.0)
    d1_ref[0, 0, :] = jnp.min(acc, axis=1)
    part2 = jnp.min(acc, axis=0)

    @pl.when(i == 0)
    def _init():
        d2_ref[0, 0, :] = part2

    @pl.when(i != 0)
    def _accum():
        d2_ref[0, 0, :] = jnp.minimum(d2_ref[0, 0, :], part2)


def kernel(xyz1, xyz2):
    B, N, D = xyz1.shape
    M = xyz2.shape[1]
    xyz2t = jnp.swapaxes(xyz2, 1, 2)
    d1, d2 = pl.pallas_call(
        _chamfer_body,
        grid=(B, N // _TN),
        in_specs=[
            pl.BlockSpec((1, _TN, D), lambda b, i: (b, i, 0)),
            pl.BlockSpec((1, D, M), lambda b, i: (b, 0, 0)),
        ],
        out_specs=[
            pl.BlockSpec((1, 1, _TN), lambda b, i: (b, 0, i)),
            pl.BlockSpec((1, 1, M), lambda b, i: (b, 0, 0)),
        ],
        out_shape=[
            jax.ShapeDtypeStruct((B, 1, N), jnp.float32),
            jax.ShapeDtypeStruct((B, 1, M), jnp.float32),
        ],
    )(xyz1, xyz2t)
    return jnp.mean(d1) + jnp.mean(d2)


# speedup vs baseline: 1.2055x; 1.2055x over previous
import jax
import jax.numpy as jnp
from jax.experimental import pallas as pl
from jax.experimental.pallas import tpu as pltpu

_TN = 512


def _chamfer_body(x1_ref, x2t_ref, d1_ref, d2_ref):
    i = pl.program_id(1)
    x1 = x1_ref[0]
    x2t = x2t_ref[0]
    sq1 = jnp.sum(x1 * x1, axis=1, keepdims=True)
    sq2 = 0.25 * jnp.sum(x2t * x2t, axis=0, keepdims=True)
    g = jax.lax.dot_general(
        x1, x2t, (((1,), (0,)), ((), ())),
        preferred_element_type=jnp.float32)
    acc = (sq1 + sq2) + g
    d1_ref[0, 0, :] = jnp.maximum(jnp.min(acc, axis=1), 0.0)
    part2 = jnp.maximum(jnp.min(acc, axis=0), 0.0)

    @pl.when(i == 0)
    def _init():
        d2_ref[0, 0, :] = part2

    @pl.when(i != 0)
    def _accum():
        d2_ref[0, 0, :] = jnp.minimum(d2_ref[0, 0, :], part2)


def kernel(xyz1, xyz2):
    B, N, D = xyz1.shape
    M = xyz2.shape[1]
    xyz2t = -2.0 * jnp.swapaxes(xyz2, 1, 2)
    d1, d2 = pl.pallas_call(
        _chamfer_body,
        grid=(B, N // _TN),
        in_specs=[
            pl.BlockSpec((1, _TN, D), lambda b, i: (b, i, 0)),
            pl.BlockSpec((1, D, M), lambda b, i: (b, 0, 0)),
        ],
        out_specs=[
            pl.BlockSpec((1, 1, _TN), lambda b, i: (b, 0, i)),
            pl.BlockSpec((1, 1, M), lambda b, i: (b, 0, 0)),
        ],
        out_shape=[
            jax.ShapeDtypeStruct((B, 1, N), jnp.float32),
            jax.ShapeDtypeStruct((B, 1, M), jnp.float32),
        ],
        compiler_params=pltpu.CompilerParams(
            dimension_semantics=("parallel", "arbitrary")),
    )(xyz1, xyz2t)
    return jnp.mean(d1) + jnp.mean(d2)


# speedup vs baseline: 1.2629x; 1.0476x over previous
import jax
import jax.numpy as jnp
from jax.experimental import pallas as pl
from jax.experimental.pallas import tpu as pltpu

_TN = 512


def _chamfer_body(a1_ref, a2t_ref, d1_ref, d2_ref):
    i = pl.program_id(1)
    a1 = a1_ref[0]
    a2t = a2t_ref[0]
    acc = jax.lax.dot_general(
        a1, a2t, (((1,), (0,)), ((), ())),
        preferred_element_type=jnp.float32)
    d1_ref[0, 0, :] = jnp.maximum(jnp.min(acc, axis=1), 0.0)
    part2 = jnp.maximum(jnp.min(acc, axis=0), 0.0)

    @pl.when(i == 0)
    def _init():
        d2_ref[0, 0, :] = part2

    @pl.when(i != 0)
    def _accum():
        d2_ref[0, 0, :] = jnp.minimum(d2_ref[0, 0, :], part2)


def kernel(xyz1, xyz2):
    B, N, D = xyz1.shape
    M = xyz2.shape[1]
    sq1 = jnp.sum(xyz1 * xyz1, axis=-1, keepdims=True)
    sq2 = jnp.sum(xyz2 * xyz2, axis=-1)[:, None, :]
    ones1 = jnp.ones((B, N, 1), jnp.float32)
    a1 = jnp.concatenate([xyz1, sq1, ones1], axis=-1)
    a2t = jnp.concatenate(
        [-2.0 * jnp.swapaxes(xyz2, 1, 2),
         jnp.ones((B, 1, M), jnp.float32), sq2], axis=1)
    d1, d2 = pl.pallas_call(
        _chamfer_body,
        grid=(B, N // _TN),
        in_specs=[
            pl.BlockSpec((1, _TN, D + 2), lambda b, i: (b, i, 0)),
            pl.BlockSpec((1, D + 2, M), lambda b, i: (b, 0, 0)),
        ],
        out_specs=[
            pl.BlockSpec((1, 1, _TN), lambda b, i: (b, 0, i)),
            pl.BlockSpec((1, 1, M), lambda b, i: (b, 0, 0)),
        ],
        out_shape=[
            jax.ShapeDtypeStruct((B, 1, N), jnp.float32),
            jax.ShapeDtypeStruct((B, 1, M), jnp.float32),
        ],
        compiler_params=pltpu.CompilerParams(
            dimension_semantics=("parallel", "arbitrary")),
    )(a1, a2t)
    return jnp.mean(d1) + jnp.mean(d2)


# speedup vs baseline: 1.2960x; 1.0262x over previous
import jax
import jax.numpy as jnp
from jax.experimental import pallas as pl
from jax.experimental.pallas import tpu as pltpu

_TN = 512


def _chamfer_body(a1_ref, a2t_ref, d1_ref, d2_ref):
    i = pl.program_id(1)
    a1 = a1_ref[0]
    a2t = a2t_ref[0]
    acc = jax.lax.dot_general(
        a1, a2t, (((1,), (0,)), ((), ())),
        preferred_element_type=jnp.float32)
    d1_ref[0, 0, :] = jnp.maximum(jnp.min(acc, axis=1), 0.0)
    part2 = jnp.maximum(jnp.min(acc, axis=0), 0.0)

    @pl.when(i == 0)
    def _init():
        d2_ref[0, 0, :] = part2

    @pl.when(i != 0)
    def _accum():
        d2_ref[0, 0, :] = jnp.minimum(d2_ref[0, 0, :], part2)


def kernel(xyz1, xyz2):
    B, N, D = xyz1.shape
    M = xyz2.shape[1]
    sq1 = jnp.sum(xyz1 * xyz1, axis=-1, keepdims=True)
    sq2 = jnp.sum(xyz2 * xyz2, axis=-1)[:, None, :]
    sq1hi = sq1.astype(jnp.bfloat16).astype(jnp.float32)
    sq1lo = sq1 - sq1hi
    sq2hi = sq2.astype(jnp.bfloat16).astype(jnp.float32)
    sq2lo = sq2 - sq2hi
    ones1 = jnp.ones((B, N, 1), jnp.float32)
    ones2 = jnp.ones((B, 1, M), jnp.float32)
    a1 = jnp.concatenate(
        [xyz1, sq1hi, sq1lo, ones1, ones1], axis=-1)
    a2t = jnp.concatenate(
        [-2.0 * jnp.swapaxes(xyz2, 1, 2),
         ones2, ones2, sq2hi, sq2lo], axis=1)
    d1, d2 = pl.pallas_call(
        _chamfer_body,
        grid=(B, N // _TN),
        in_specs=[
            pl.BlockSpec((1, _TN, D + 4), lambda b, i: (b, i, 0)),
            pl.BlockSpec((1, D + 4, M), lambda b, i: (b, 0, 0)),
        ],
        out_specs=[
            pl.BlockSpec((1, 1, _TN), lambda b, i: (b, 0, i)),
            pl.BlockSpec((1, 1, M), lambda b, i: (b, 0, 0)),
        ],
        out_shape=[
            jax.ShapeDtypeStruct((B, 1, N), jnp.float32),
            jax.ShapeDtypeStruct((B, 1, M), jnp.float32),
        ],
        compiler_params=pltpu.CompilerParams(
            dimension_semantics=("parallel", "arbitrary")),
    )(a1, a2t)
    return jnp.mean(d1) + jnp.mean(d2)
